# Initial kernel scaffold; baseline (speedup 1.0000x reference)
import jax
import jax.numpy as jnp
from jax.experimental import pallas as pl


def kernel(probas, labels):
    raise NotImplementedError("write your pallas kernel here")


# speedup vs baseline: 27.1047x; 27.1047x over previous
import functools

import jax
import jax.numpy as jnp
from jax import lax
from jax.experimental import pallas as pl
from jax.experimental.pallas import tpu as pltpu
from jax.experimental.pallas import tpu_sc as plsc

K = 2048
NC, NS, L = 2, 16, 16
NW = NC * NS
CHUNK = 1024


def _sc_hist(P, C):
    PW = P // NW
    NCH = PW // CHUNK
    GRP = CHUNK // L

    mesh = plsc.VectorSubcoreMesh(core_axis_name="c", subcore_axis_name="s")

    @functools.partial(
        pl.kernel,
        out_type=(
            jax.ShapeDtypeStruct((NW, C, K), jnp.int32),
            jax.ShapeDtypeStruct((NW, C, K), jnp.int32),
        ),
        mesh=mesh,
        scratch_types=[
            pltpu.VMEM((CHUNK * C,), jnp.float32),
            pltpu.VMEM((CHUNK,), jnp.int32),
            pltpu.VMEM((C, K), jnp.int32),
            pltpu.VMEM((C, K), jnp.int32),
        ],
        compiler_params=pltpu.CompilerParams(needs_layout_passes=False),
    )
    def kern(probas_hbm, labels_hbm, hall_hbm, hfg_hbm, pbuf, lbuf, ha, hf):
        wid = lax.axis_index("s") * NC + lax.axis_index("c")
        zeros16 = jnp.zeros((L,), jnp.int32)

        def zbody(j, _):
            for c in range(C):
                ha[c, pl.ds(j * L, L)] = zeros16
                hf[c, pl.ds(j * L, L)] = zeros16
            return 0

        lax.fori_loop(0, K // L, zbody, 0)

        lanesC = lax.iota(jnp.int32, L) * C
        ones16 = jnp.ones((L,), jnp.int32)
        kf = jnp.float32(K)
        kmax = jnp.full((L,), K - 1, jnp.int32)

        def chunk_body(t, _):
            pt0 = wid * PW + t * CHUNK
            pltpu.sync_copy(probas_hbm.at[pl.ds(pt0 * C, CHUNK * C)], pbuf)
            pltpu.sync_copy(labels_hbm.at[pl.ds(pt0, CHUNK)], lbuf)

            def grp_body(g, _):
                lbl = lbuf[pl.ds(g * L, L)]
                idx0 = g * (L * C) + lanesC
                pfg = plsc.load_gather(pbuf, [idx0 + lbl])
                bfg = jnp.minimum((pfg * kf).astype(jnp.int32), kmax)
                plsc.addupdate_scatter(hf, [lbl, bfg], ones16)
                for c in range(C):
                    p = plsc.load_gather(pbuf, [idx0 + c])
                    b = jnp.minimum((p * kf).astype(jnp.int32), kmax)
                    plsc.addupdate_scatter(
                        ha, [jnp.full((L,), c, jnp.int32), b], ones16
                    )
                return 0

            lax.fori_loop(0, GRP, grp_body, 0)
            return 0

        lax.fori_loop(0, NCH, chunk_body, 0)
        pltpu.sync_copy(ha, hall_hbm.at[wid])
        pltpu.sync_copy(hf, hfg_hbm.at[wid])

    return kern


def _tc_phase2(C):
    TK = 256

    def kern(hall_ref, hfg_ref, out_ref):
        hall = jnp.sum(hall_ref[...].astype(jnp.float32), axis=0)
        hfg = jnp.sum(hfg_ref[...].astype(jnp.float32), axis=0)
        hbg = hall - hfg
        gts = jnp.sum(hfg, axis=1, keepdims=True)
        jrow = lax.broadcasted_iota(jnp.int32, (K, TK), 0)
        kcol = lax.broadcasted_iota(jnp.int32, (K, TK), 1)
        total = jnp.float32(0.0)
        f0sum = jnp.float32(0.0)
        for tile in range(K // TK):
            k0 = tile * TK
            m1 = jnp.where(jrow + kcol + k0 < K, 1.0, 0.0)
            m2 = jnp.where(jrow >= kcol + k0, 1.0, 0.0)
            a = jnp.dot(hfg, m1, preferred_element_type=jnp.float32)
            b = jnp.dot(hbg, m2, preferred_element_type=jnp.float32)
            denom = gts + b
            F = jnp.where(
                denom > 0.0,
                1.0 - (gts - a) / jnp.where(denom > 0.0, denom, 1.0),
                0.0,
            )
            total = total + jnp.sum(F)
            if tile == 0:
                f0sum = jnp.sum(F[:, 0:1])
        out_ref[0, 0] = (total - 0.5 * f0sum) / jnp.float32(C * K)

    return kern


def kernel(probas, labels):
    P, C = probas.shape
    hall, hfg = _sc_hist(P, C)(probas.reshape(-1), labels.reshape(-1))
    out = pl.pallas_call(
        _tc_phase2(C),
        out_shape=jax.ShapeDtypeStruct((1, 1), jnp.float32),
        out_specs=pl.BlockSpec(memory_space=pltpu.SMEM),
    )(hall, hfg)
    return out[0, 0]
